# baseline (speedup 1.0000x reference)
import jax
import jax.numpy as jnp
from jax import lax
from jax.experimental import pallas as pl
from jax.experimental.pallas import tpu as pltpu
from jax.experimental.pallas import tpu_sc as plsc

N = 10000
E = 320000
D = 128
NC = 2
NS = 16
NW = NC * NS
S = 2
ES = E // S
CH = 50
CPW = ES // (CH * NW)
G = 10
PC = CPW // G
P = PC // 2
NP = 10112
RPT = NP // NS

_LOG2E = 1.4426950408889634
_LN2 = 0.6931471805599453


def _ssp_fast(v):
    p = jnp.exp2(v * _LOG2E)
    return (jnp.log2(1.0 + p) - 1.0) * _LN2


def _fmlp_body(ea, w1t, b1, w2t, b2, out):
    h = jnp.dot(ea[...].astype(jnp.bfloat16), w1t[...],
                preferred_element_type=jnp.float32) + b1[...]
    h = _ssp_fast(h)
    h = jnp.dot(h.astype(jnp.bfloat16), w2t[...],
                preferred_element_type=jnp.float32) + b2[...]
    out[...] = _ssp_fast(h)


def _xw_body(x, fc1t, out):
    out[...] = jnp.dot(x[...], fc1t[...], preferred_element_type=jnp.float32)


def _state_body(agg2, w1t, w2t, out):
    a = agg2[0] + agg2[1]
    h = _ssp_fast(jnp.dot(a, w1t[...], preferred_element_type=jnp.float32))
    out[...] = jnp.dot(h, w2t[...], preferred_element_type=jnp.float32)


def _filter_mlp_slab(edge_attr, w1t, b1, w2t, b2, slab):
    BE = 2000
    off = slab * (ES // BE)
    return pl.pallas_call(
        _fmlp_body,
        grid=(ES // BE,),
        in_specs=[
            pl.BlockSpec((BE, D), lambda i: (off + i, 0)),
            pl.BlockSpec((D, D), lambda i: (0, 0)),
            pl.BlockSpec((1, D), lambda i: (0, 0)),
            pl.BlockSpec((D, D), lambda i: (0, 0)),
            pl.BlockSpec((1, D), lambda i: (0, 0)),
        ],
        out_specs=pl.BlockSpec((BE, D), lambda i: (i, 0)),
        out_shape=jax.ShapeDtypeStruct((ES, D), jnp.float32),
    )(edge_attr, w1t, b1, w2t, b2)


def _xw(x, fc1t):
    BN = 1000
    return pl.pallas_call(
        _xw_body,
        grid=(N // BN,),
        in_specs=[
            pl.BlockSpec((BN, D), lambda i: (i, 0)),
            pl.BlockSpec((D, D), lambda i: (0, 0)),
        ],
        out_specs=pl.BlockSpec((BN, D), lambda i: (i, 0)),
        out_shape=jax.ShapeDtypeStruct((N, D), jnp.float32),
    )(x, fc1t)


def _state_mlp(agg2, w1t, w2t):
    BN = 1000
    return pl.pallas_call(
        _state_body,
        grid=(N // BN,),
        in_specs=[
            pl.BlockSpec((NC, BN, D), lambda i: (0, i, 0)),
            pl.BlockSpec((D, D), lambda i: (0, 0)),
            pl.BlockSpec((D, D), lambda i: (0, 0)),
        ],
        out_specs=pl.BlockSpec((BN, D), lambda i: (i, 0)),
        out_shape=jax.ShapeDtypeStruct((N, D), jnp.float32),
    )(agg2, w1t, w2t)


def _sc_body(xw_hbm, w_hbm, src_hbm, dst_hbm, eidx_hbm, prev_hbm, out_hbm,
             idx_s, idx_d, idx_e, wb0, wb1, rb0, rb1, pb0, pb1,
             sg0, sg1, sw0, sw1, ss0, ss1, agg_sh):
    c = lax.axis_index("c")
    t = lax.axis_index("s")
    wid = c * NS + t
    wb = (wb0, wb1)
    rb = (rb0, rb1)
    pb = (pb0, pb1)
    sg = (sg0, sg1)
    sw = (sw0, sw1)
    ss = (ss0, ss1)

    pltpu.sync_copy(prev_hbm.at[c, pl.ds(t * RPT, RPT)],
                    agg_sh.at[pl.ds(t * RPT, RPT)])
    plsc.subcore_barrier()

    def start_gather(j, b):
        pltpu.async_copy(w_hbm.at[idx_e.at[j]], wb[b], sw[b])
        pltpu.async_copy(xw_hbm.at[idx_s.at[j]], rb[b], sg[b])

    def wait_gather(b):
        pltpu.make_async_copy(w_hbm.at[idx_e.at[0]], wb[b], sw[b]).wait()
        pltpu.make_async_copy(xw_hbm.at[idx_s.at[0]], rb[b], sg[b]).wait()

    def mul(b):
        def mrow(r, u):
            for k in range(D // 16):
                sl = pl.ds(k * 16, 16)
                pb[b][r, sl] = rb[b][r, sl] * wb[b][r, sl]
            return u
        lax.fori_loop(0, CH, mrow, 0)

    def start_scatter(j, b):
        pltpu.async_copy(pb[b], agg_sh.at[idx_d.at[j]], ss[b], add=True)

    def wait_scatter(b):
        pltpu.make_async_copy(pb[b], agg_sh.at[idx_d.at[0]], ss[b]).wait()

    def half(i, b, first, last):
        wait_gather(b)
        if not first:
            wait_scatter(b)
        mul(b)
        if not last:
            start_gather(i + 2, b)
        start_scatter(i, b)

    def pair(p, first, last):
        half(2 * p, 0, first, last)
        half(2 * p + 1, 1, first, last)

    def group(g, carry):
        pltpu.sync_copy(src_hbm.at[wid, g], idx_s)
        pltpu.sync_copy(dst_hbm.at[wid, g], idx_d)
        pltpu.sync_copy(eidx_hbm.at[wid, g], idx_e)

        start_gather(0, 0)
        start_gather(1, 1)
        pair(0, True, False)

        def steady(p, u):
            pair(p, False, False)
            return u
        lax.fori_loop(1, P - 1, steady, 0)
        pair(P - 1, False, True)
        wait_scatter(0)
        wait_scatter(1)
        return carry
    lax.fori_loop(0, G, group, 0)

    plsc.subcore_barrier()
    pltpu.sync_copy(agg_sh.at[pl.ds(t * RPT, RPT)],
                    out_hbm.at[c, pl.ds(t * RPT, RPT)])


def _scatter_slab(xw, w, srcs, dsts, eidx, prev):
    mesh = plsc.VectorSubcoreMesh(core_axis_name="c", subcore_axis_name="s")
    f = pl.kernel(
        _sc_body,
        out_type=jax.ShapeDtypeStruct((NC, NP, D), jnp.float32),
        mesh=mesh,
        scratch_types=[
            pltpu.VMEM((PC, CH), jnp.int32),
            pltpu.VMEM((PC, CH), jnp.int32),
            pltpu.VMEM((PC, CH), jnp.int32),
            pltpu.VMEM((CH, D), jnp.float32),
            pltpu.VMEM((CH, D), jnp.float32),
            pltpu.VMEM((CH, D), jnp.float32),
            pltpu.VMEM((CH, D), jnp.float32),
            pltpu.VMEM((CH, D), jnp.float32),
            pltpu.VMEM((CH, D), jnp.float32),
            pltpu.SemaphoreType.DMA,
            pltpu.SemaphoreType.DMA,
            pltpu.SemaphoreType.DMA,
            pltpu.SemaphoreType.DMA,
            pltpu.SemaphoreType.DMA,
            pltpu.SemaphoreType.DMA,
            pltpu.VMEM_SHARED((NP, D), jnp.float32),
        ],
    )
    return f(xw, w, srcs, dsts, eidx, prev)


def kernel(x, edge_index, edge_attr, fc1_w, fmlp_w1, fmlp_b1, fmlp_w2,
           fmlp_b2, st_w1, st_w2):
    srcs = edge_index[0].astype(jnp.int32).reshape(S, NW, G, PC, CH)
    dsts = edge_index[1].astype(jnp.int32).reshape(S, NW, G, PC, CH)
    eidx = jnp.arange(ES, dtype=jnp.int32).reshape(NW, G, PC, CH)

    w1t = fmlp_w1.T.astype(jnp.bfloat16)
    w2t = fmlp_w2.T.astype(jnp.bfloat16)
    ws = [_filter_mlp_slab(edge_attr, w1t, fmlp_b1.reshape(1, D),
                           w2t, fmlp_b2.reshape(1, D), s) for s in range(S)]
    xw = _xw(x, fc1_w.T)
    agg = jnp.zeros((NC, NP, D), jnp.float32)
    for s in range(S):
        agg = _scatter_slab(xw, ws[s], srcs[s], dsts[s], eidx, agg)
    return _state_mlp(agg, st_w1.T, st_w2.T)


# speedup vs baseline: 1.0498x; 1.0498x over previous
import jax
import jax.numpy as jnp
from jax import lax
from jax.experimental import pallas as pl
from jax.experimental.pallas import tpu as pltpu
from jax.experimental.pallas import tpu_sc as plsc

N = 10000
E = 320000
D = 128
NC = 2
NS = 16
NW = NC * NS
S = 2
ES = E // S
CH = 40
CPW = ES // (CH * NW)
G = 5
PC = CPW // G
P = PC // 2
NP = 10112
RPT = NP // NS

_LOG2E = 1.4426950408889634
_LN2 = 0.6931471805599453


def _ssp_fast(v):
    p = jnp.exp2(v * _LOG2E)
    return (jnp.log2(1.0 + p) - 1.0) * _LN2


def _fmlp_body(ea, w1t, b1, w2t, b2, out):
    h = jnp.dot(ea[...].astype(jnp.bfloat16), w1t[...],
                preferred_element_type=jnp.float32) + b1[...]
    h = _ssp_fast(h)
    h = jnp.dot(h.astype(jnp.bfloat16), w2t[...],
                preferred_element_type=jnp.float32) + b2[...]
    out[...] = _ssp_fast(h)


def _xw_body(x, fc1t, out):
    out[...] = jnp.dot(x[...], fc1t[...], preferred_element_type=jnp.float32)


def _state_body(agg2, w1t, w2t, out):
    a = agg2[0] + agg2[1]
    h = _ssp_fast(jnp.dot(a, w1t[...], preferred_element_type=jnp.float32))
    out[...] = jnp.dot(h, w2t[...], preferred_element_type=jnp.float32)


def _filter_mlp_slab(edge_attr, w1t, b1, w2t, b2, slab):
    BE = 2000
    off = slab * (ES // BE)
    return pl.pallas_call(
        _fmlp_body,
        grid=(ES // BE,),
        in_specs=[
            pl.BlockSpec((BE, D), lambda i: (off + i, 0)),
            pl.BlockSpec((D, D), lambda i: (0, 0)),
            pl.BlockSpec((1, D), lambda i: (0, 0)),
            pl.BlockSpec((D, D), lambda i: (0, 0)),
            pl.BlockSpec((1, D), lambda i: (0, 0)),
        ],
        out_specs=pl.BlockSpec((BE, D), lambda i: (i, 0)),
        out_shape=jax.ShapeDtypeStruct((ES, D), jnp.float32),
    )(edge_attr, w1t, b1, w2t, b2)


def _xw(x, fc1t):
    BN = 1000
    return pl.pallas_call(
        _xw_body,
        grid=(N // BN,),
        in_specs=[
            pl.BlockSpec((BN, D), lambda i: (i, 0)),
            pl.BlockSpec((D, D), lambda i: (0, 0)),
        ],
        out_specs=pl.BlockSpec((BN, D), lambda i: (i, 0)),
        out_shape=jax.ShapeDtypeStruct((N, D), jnp.float32),
    )(x, fc1t)


def _state_mlp(agg2, w1t, w2t):
    BN = 1000
    return pl.pallas_call(
        _state_body,
        grid=(N // BN,),
        in_specs=[
            pl.BlockSpec((NC, BN, D), lambda i: (0, i, 0)),
            pl.BlockSpec((D, D), lambda i: (0, 0)),
            pl.BlockSpec((D, D), lambda i: (0, 0)),
        ],
        out_specs=pl.BlockSpec((BN, D), lambda i: (i, 0)),
        out_shape=jax.ShapeDtypeStruct((N, D), jnp.float32),
    )(agg2, w1t, w2t)


def _sc_body(xw_hbm, w_hbm, src_hbm, dst_hbm, prev_hbm, out_hbm,
             idx_s, idx_d, wb0, wb1, rb0, rb1, pb0, pb1,
             sg0, sg1, sw0, sw1, ss0, ss1, agg_sh):
    c = lax.axis_index("c")
    t = lax.axis_index("s")
    wid = c * NS + t
    wb = (wb0, wb1)
    rb = (rb0, rb1)
    pb = (pb0, pb1)
    sg = (sg0, sg1)
    sw = (sw0, sw1)
    ss = (ss0, ss1)

    pltpu.sync_copy(prev_hbm.at[c, pl.ds(t * RPT, RPT)],
                    agg_sh.at[pl.ds(t * RPT, RPT)])
    plsc.subcore_barrier()

    def mul(b):
        def mrow(r, u):
            for k in range(D // 16):
                sl = pl.ds(k * 16, 16)
                pb[b][r, sl] = rb[b][r, sl] * wb[b][r, sl]
            return u
        lax.fori_loop(0, CH, mrow, 0)

    def group(g, carry):
        pltpu.sync_copy(src_hbm.at[wid, g], idx_s)
        pltpu.sync_copy(dst_hbm.at[wid, g], idx_d)

        def start_gather(j, b):
            base = (wid * (ES // NW // 8) + (g * PC + j) * (CH // 8)) * 8
            pltpu.async_copy(w_hbm.at[pl.ds(base, CH)], wb[b], sw[b])
            pltpu.async_copy(xw_hbm.at[idx_s.at[j]], rb[b], sg[b])

        def wait_gather(b):
            pltpu.make_async_copy(w_hbm.at[pl.ds(0, CH)], wb[b], sw[b]).wait()
            pltpu.make_async_copy(xw_hbm.at[idx_s.at[0]], rb[b], sg[b]).wait()

        def start_scatter(j, b):
            pltpu.async_copy(pb[b], agg_sh.at[idx_d.at[j]], ss[b], add=True)

        def wait_scatter(b):
            pltpu.make_async_copy(pb[b], agg_sh.at[idx_d.at[0]], ss[b]).wait()

        def half(i, b, first, lastg):
            wait_gather(b)
            if not first:
                wait_scatter(b)
            mul(b)
            if not lastg:
                start_gather(i + 2, b)
            start_scatter(i, b)

        start_gather(0, 0)
        start_gather(1, 1)
        half(0, 0, True, False)
        half(1, 1, True, False)

        def steady(p, u):
            half(2 * p, 0, False, False)
            half(2 * p + 1, 1, False, False)
            return u
        lax.fori_loop(1, P - 1, steady, 0)
        half(2 * P - 2, 0, False, False)
        half(2 * P - 1, 1, False, True)
        half(2 * P, 0, False, True)
        wait_scatter(0)
        wait_scatter(1)
        return carry
    lax.fori_loop(0, G, group, 0)

    plsc.subcore_barrier()
    pltpu.sync_copy(agg_sh.at[pl.ds(t * RPT, RPT)],
                    out_hbm.at[c, pl.ds(t * RPT, RPT)])


def _scatter_slab(xw, w, srcs, dsts, prev):
    mesh = plsc.VectorSubcoreMesh(core_axis_name="c", subcore_axis_name="s")
    f = pl.kernel(
        _sc_body,
        out_type=jax.ShapeDtypeStruct((NC, NP, D), jnp.float32),
        mesh=mesh,
        scratch_types=[
            pltpu.VMEM((PC, CH), jnp.int32),
            pltpu.VMEM((PC, CH), jnp.int32),
            pltpu.VMEM((CH, D), jnp.float32),
            pltpu.VMEM((CH, D), jnp.float32),
            pltpu.VMEM((CH, D), jnp.float32),
            pltpu.VMEM((CH, D), jnp.float32),
            pltpu.VMEM((CH, D), jnp.float32),
            pltpu.VMEM((CH, D), jnp.float32),
            pltpu.SemaphoreType.DMA,
            pltpu.SemaphoreType.DMA,
            pltpu.SemaphoreType.DMA,
            pltpu.SemaphoreType.DMA,
            pltpu.SemaphoreType.DMA,
            pltpu.SemaphoreType.DMA,
            pltpu.VMEM_SHARED((NP, D), jnp.float32),
        ],
    )
    return f(xw, w, srcs, dsts, prev)


def kernel(x, edge_index, edge_attr, fc1_w, fmlp_w1, fmlp_b1, fmlp_w2,
           fmlp_b2, st_w1, st_w2):
    srcs = edge_index[0].astype(jnp.int32).reshape(S, NW, G, PC, CH)
    dsts = edge_index[1].astype(jnp.int32).reshape(S, NW, G, PC, CH)

    w1t = fmlp_w1.T.astype(jnp.bfloat16)
    w2t = fmlp_w2.T.astype(jnp.bfloat16)
    ws = [_filter_mlp_slab(edge_attr, w1t, fmlp_b1.reshape(1, D),
                           w2t, fmlp_b2.reshape(1, D), s) for s in range(S)]
    xw = _xw(x, fc1_w.T)
    agg = jnp.zeros((NC, NP, D), jnp.float32)
    for s in range(S):
        agg = _scatter_slab(xw, ws[s], srcs[s], dsts[s], agg)
    return _state_mlp(agg, st_w1.T, st_w2.T)
